# Initial kernel scaffold; baseline (speedup 1.0000x reference)
import jax
import jax.numpy as jnp
from jax.experimental import pallas as pl


def kernel(tokens_idx, emb_weight):
    raise NotImplementedError("write your pallas kernel here")


# speedup vs baseline: 2.5148x; 2.5148x over previous
import jax
import jax.numpy as jnp
from jax import lax
from jax.experimental import pallas as pl
from jax.experimental.pallas import tpu as pltpu
from jax.experimental.pallas import tpu_sc as plsc

NC, NS = 2, 16
NW = NC * NS
B, L, DIM = 16384, 50, 32
BAGS_W = B // NW
CB = 32
NCH = BAGS_W // CB
RPC = CB * L
INV_L = 1.0 / L


def _body(tok_hbm, tab_hbm, out_hbm, idx_v, rows_v, out_v, sem):
    wid = lax.axis_index("s") * NC + lax.axis_index("c")

    def chunk(c, carry):
        bag0 = wid * BAGS_W + c * CB
        pltpu.sync_copy(tok_hbm.at[pl.ds(bag0 * L, RPC)], idx_v)
        pltpu.async_copy(tab_hbm.at[idx_v], rows_v, sem).wait()

        def bag(b, carry2):
            def row(r, acc):
                a0, a1 = acc
                i = b * L + r
                return (a0 + rows_v[i, 0:16], a1 + rows_v[i, 16:32])

            z = jnp.zeros((16,), jnp.float32)
            a0, a1 = lax.fori_loop(0, L, row, (z, z))
            out_v[b, 0:16] = a0 * INV_L
            out_v[b, 16:32] = a1 * INV_L
            return carry2

        lax.fori_loop(0, CB, bag, 0)
        pltpu.sync_copy(out_v, out_hbm.at[pl.ds(bag0, CB)])
        return carry

    lax.fori_loop(0, NCH, chunk, 0)


def kernel(tokens_idx, emb_weight):
    tok = tokens_idx.reshape(-1).astype(jnp.int32)
    mesh = plsc.VectorSubcoreMesh(core_axis_name="c", subcore_axis_name="s")
    f = pl.kernel(
        _body,
        out_type=jax.ShapeDtypeStruct((B, DIM), jnp.float32),
        mesh=mesh,
        compiler_params=pltpu.CompilerParams(use_tc_tiling_on_sc=False),
        scratch_types=[
            pltpu.VMEM((RPC,), jnp.int32),
            pltpu.VMEM((RPC, DIM), jnp.float32),
            pltpu.VMEM((CB, DIM), jnp.float32),
            pltpu.SemaphoreType.DMA,
        ],
    )
    return f(tok, emb_weight)


# speedup vs baseline: 2.9339x; 1.1667x over previous
import jax
import jax.numpy as jnp
from jax import lax
from jax.experimental import pallas as pl
from jax.experimental.pallas import tpu as pltpu
from jax.experimental.pallas import tpu_sc as plsc

NC, NS = 2, 16
NW = NC * NS
B, L, DIM = 16384, 50, 32
BAGS_W = B // NW
CB = 32
NCH = BAGS_W // CB
NST = NCH // 2
RPC = CB * L
INV_L = 1.0 / L


def _body(tok_hbm, tab_hbm, out_hbm,
          idx0, idx1, rows0, rows1, out0, out1, sem0, sem1):
    wid = lax.axis_index("s") * NC + lax.axis_index("c")
    idx = (idx0, idx1)
    rows = (rows0, rows1)
    outs = (out0, out1)
    sems = (sem0, sem1)

    def fire(c, p):
        bag0 = wid * BAGS_W + c * CB
        pltpu.sync_copy(tok_hbm.at[pl.ds(bag0 * L, RPC)], idx[p])
        pltpu.async_copy(tab_hbm.at[idx[p]], rows[p], sems[p])

    def process(c, p):
        pltpu.make_async_copy(tab_hbm.at[idx[p]], rows[p], sems[p]).wait()
        rv = rows[p]
        ov = outs[p]

        def bag(b, carry):
            a0 = rv[b * L, 0:16]
            a1 = rv[b * L, 16:32]
            for r in range(1, L):
                a0 = a0 + rv[b * L + r, 0:16]
                a1 = a1 + rv[b * L + r, 16:32]
            ov[b, 0:16] = a0 * INV_L
            ov[b, 16:32] = a1 * INV_L
            return carry

        lax.fori_loop(0, CB, bag, 0)
        bag0 = wid * BAGS_W + c * CB
        pltpu.sync_copy(ov, out_hbm.at[pl.ds(bag0, CB)])

    fire(0, 0)

    def step(s, carry):
        c0 = s * 2
        fire(c0 + 1, 1)
        process(c0, 0)

        @pl.when(s < NST - 1)
        def _():
            fire(c0 + 2, 0)

        process(c0 + 1, 1)
        return carry

    lax.fori_loop(0, NST, step, 0)


def kernel(tokens_idx, emb_weight):
    tok = tokens_idx.reshape(-1).astype(jnp.int32)
    mesh = plsc.VectorSubcoreMesh(core_axis_name="c", subcore_axis_name="s")
    f = pl.kernel(
        _body,
        out_type=jax.ShapeDtypeStruct((B, DIM), jnp.float32),
        mesh=mesh,
        compiler_params=pltpu.CompilerParams(use_tc_tiling_on_sc=False),
        scratch_types=[
            pltpu.VMEM((RPC,), jnp.int32),
            pltpu.VMEM((RPC,), jnp.int32),
            pltpu.VMEM((RPC, DIM), jnp.float32),
            pltpu.VMEM((RPC, DIM), jnp.float32),
            pltpu.VMEM((CB, DIM), jnp.float32),
            pltpu.VMEM((CB, DIM), jnp.float32),
            pltpu.SemaphoreType.DMA,
            pltpu.SemaphoreType.DMA,
        ],
    )
    return f(tok, emb_weight)
